# baseline (speedup 1.0000x reference)
import functools

import jax
import jax.numpy as jnp
from jax import lax
from jax.experimental import pallas as pl
from jax.experimental.pallas import tpu as pltpu
from jax.experimental.pallas import tpu_sc as plsc

BATCH = 16384
DIM = 32
NC = 2
NS = 16
NW = NC * NS
B_PER_W = BATCH // NW
CHUNK = 128
NCHUNK = B_PER_W // CHUNK
WEIGHT_DECAY = 0.025


def _sc_gather(u2d, i2d, j2d, W, H):
    mesh = plsc.VectorSubcoreMesh(core_axis_name="c", subcore_axis_name="s")
    out = jax.ShapeDtypeStruct((BATCH, DIM), jnp.float32)

    @functools.partial(
        pl.kernel,
        mesh=mesh,
        out_type=(out, out, out),
        compiler_params=pltpu.CompilerParams(use_tc_tiling_on_sc=False),
        scratch_types=[
            pltpu.VMEM((NCHUNK, CHUNK), jnp.int32),
            pltpu.VMEM((NCHUNK, CHUNK), jnp.int32),
            pltpu.VMEM((NCHUNK, CHUNK), jnp.int32),
            pltpu.VMEM((B_PER_W, DIM), jnp.float32),
            pltpu.VMEM((B_PER_W, DIM), jnp.float32),
            pltpu.VMEM((B_PER_W, DIM), jnp.float32),
            pltpu.SemaphoreType.DMA,
        ],
    )
    def k(u_hbm, i_hbm, j_hbm, w_hbm, h_hbm, ou_hbm, oi_hbm, oj_hbm,
          uix, iix, jix, urows, irows, jrows, sem):
        wid = lax.axis_index("s") * NC + lax.axis_index("c")
        base = wid * B_PER_W
        row0 = wid * NCHUNK
        pltpu.sync_copy(u_hbm.at[pl.ds(row0, NCHUNK)], uix)
        pltpu.sync_copy(i_hbm.at[pl.ds(row0, NCHUNK)], iix)
        pltpu.sync_copy(j_hbm.at[pl.ds(row0, NCHUNK)], jix)
        copies = []
        for c in range(NCHUNK):
            dst = pl.ds(c * CHUNK, CHUNK)
            copies.append(pltpu.async_copy(w_hbm.at[uix.at[c]], urows.at[dst], sem))
            copies.append(pltpu.async_copy(h_hbm.at[iix.at[c]], irows.at[dst], sem))
            copies.append(pltpu.async_copy(h_hbm.at[jix.at[c]], jrows.at[dst], sem))
        for cp in copies:
            cp.wait()
        pltpu.sync_copy(urows, ou_hbm.at[pl.ds(base, B_PER_W)])
        pltpu.sync_copy(irows, oi_hbm.at[pl.ds(base, B_PER_W)])
        pltpu.sync_copy(jrows, oj_hbm.at[pl.ds(base, B_PER_W)])

    return k(u2d, i2d, j2d, W, H)


def _tc_loss_body(u_ref, i_ref, j_ref, loss_ref, reg_ref):
    u = u_ref[...]
    hi = i_ref[...]
    hj = j_ref[...]
    x_ui = jnp.sum(u * hi, axis=1)
    x_uj = jnp.sum(u * hj, axis=1)
    x_uij = jnp.clip(x_ui - x_uj, -80.0, 100000000.0)
    z = -x_uij
    softplus = jnp.maximum(z, 0.0) + jnp.log1p(jnp.exp(-jnp.abs(z)))
    log_prob = jnp.sum(softplus)
    reg = WEIGHT_DECAY * (jnp.sum(u * u) + jnp.sum(hi * hi) + jnp.sum(hj * hj))
    loss_ref[0, 0] = log_prob + reg
    reg_ref[0, 0] = reg


def _tc_loss(u_raw, i_raw, j_raw):
    scalar = jax.ShapeDtypeStruct((1, 1), jnp.float32)
    return pl.pallas_call(
        _tc_loss_body,
        out_shape=(scalar, scalar),
        out_specs=(pl.BlockSpec(memory_space=pltpu.SMEM),
                   pl.BlockSpec(memory_space=pltpu.SMEM)),
    )(u_raw, i_raw, j_raw)


def kernel(u, i, j, adv, W, H):
    shape2d = (BATCH // CHUNK, CHUNK)
    u_raw, i_raw, j_raw = _sc_gather(
        u.reshape(shape2d), i.reshape(shape2d), j.reshape(shape2d), W, H)
    loss, reg = _tc_loss(u_raw, i_raw, j_raw)
    total = loss[0, 0]
    if adv is True:
        total = total + reg[0, 0]
    return total
